# baseline (speedup 1.0000x reference)
import functools

import jax
import jax.numpy as jnp
from jax import lax
from jax.experimental import pallas as pl
from jax.experimental.pallas import tpu as pltpu
from jax.experimental.pallas import tpu_sc as plsc

N = 10000
E = 160000
IN_FEATS = 128
OUT_FEATS = 128
MEM = 8
NUM_RELS = 5
RM = NUM_RELS * MEM

NC = 2
NS = 16
NW = NC * NS
CH = 128
EP = 163840
EPW = EP // NW
KCH = EPW // CH
W32 = 64
K0CH = 18
K1CH = 62
EPWMAX = K1CH * CH
NP = 10240
NPT = NP // NS

@functools.lru_cache(maxsize=None)
def _build_sc_gather():
    mesh = plsc.VectorSubcoreMesh(
        core_axis_name="c", subcore_axis_name="s",
        num_cores=NC, num_subcores=NS)

    @functools.partial(
        pl.kernel,
        out_type=(jax.ShapeDtypeStruct((EP, IN_FEATS), jnp.float32),
                  jax.ShapeDtypeStruct((EP, IN_FEATS), jnp.float32)),
        mesh=mesh,
        scratch_types=[
            pltpu.VMEM((EPWMAX,), jnp.int32),
            pltpu.VMEM((EPWMAX,), jnp.int32),
            pltpu.VMEM((2, CH, IN_FEATS), jnp.float32),
            pltpu.VMEM((2, CH, IN_FEATS), jnp.float32),
            pltpu.SemaphoreType.DMA,
            pltpu.SemaphoreType.DMA,
            pltpu.SemaphoreType.DMA,
            pltpu.SemaphoreType.DMA,
        ],
    )
    def _sc_gather(x_hbm, src_hbm, dst_hbm, hs_hbm, hd_hbm,
                   si_v, di_v, sr_v, dr_v, ss0, ss1, sd0, sd1):
        cid = lax.axis_index("c")
        sid = lax.axis_index("s")
        my_k = jnp.where(cid == 0, K0CH, K1CH)
        base_c = jnp.where(cid == 0, sid * K0CH, NS * K0CH + sid * K1CH)
        base = pl.multiple_of(base_c * CH, 8)
        my_e = my_k * CH
        pltpu.sync_copy(src_hbm.at[pl.ds(base, EPWMAX)], si_v)
        pltpu.sync_copy(dst_hbm.at[pl.ds(base, EPWMAX)], di_v)

        def body(k2, carry):
            o0 = pl.multiple_of(base + (2 * k2) * CH, 8)
            o1 = pl.multiple_of(base + (2 * k2 + 1) * CH, 8)
            c0 = 2 * k2 * CH
            c1 = (2 * k2 + 1) * CH
            cs0 = pltpu.async_copy(x_hbm.at[si_v.at[pl.ds(c0, CH)]],
                                   sr_v.at[0], ss0)
            cd0 = pltpu.async_copy(x_hbm.at[di_v.at[pl.ds(c0, CH)]],
                                   dr_v.at[0], sd0)
            cs1 = pltpu.async_copy(x_hbm.at[si_v.at[pl.ds(c1, CH)]],
                                   sr_v.at[1], ss1)
            cd1 = pltpu.async_copy(x_hbm.at[di_v.at[pl.ds(c1, CH)]],
                                   dr_v.at[1], sd1)
            cs0.wait()
            pltpu.sync_copy(sr_v.at[0], hs_hbm.at[pl.ds(o0, CH)])
            cd0.wait()
            pltpu.sync_copy(dr_v.at[0], hd_hbm.at[pl.ds(o0, CH)])
            cs1.wait()
            pltpu.sync_copy(sr_v.at[1], hs_hbm.at[pl.ds(o1, CH)])
            cd1.wait()
            pltpu.sync_copy(dr_v.at[1], hd_hbm.at[pl.ds(o1, CH)])
            return carry

        lax.fori_loop(0, my_k // 2, body, 0)

    return _sc_gather


@functools.lru_cache(maxsize=None)
def _build_sc_scatter():
    mesh = plsc.VectorSubcoreMesh(
        core_axis_name="c", subcore_axis_name="s",
        num_cores=NC, num_subcores=NS)

    @functools.partial(
        pl.kernel,
        out_type=(jax.ShapeDtypeStruct((NC * NP, OUT_FEATS), jnp.float32),
                  jax.ShapeDtypeStruct((NW * NP,), jnp.float32)),
        mesh=mesh,
        compiler_params=pltpu.CompilerParams(needs_layout_passes=False),
        scratch_types=[
            pltpu.VMEM((CH,), jnp.int32),
            pltpu.VMEM((CH, OUT_FEATS), jnp.float32),
            pltpu.VMEM((NP,), jnp.float32),
            pltpu.VMEM_SHARED((NP, OUT_FEATS), jnp.float32),
        ],
    )
    def _sc_scatter(msg_hbm, dst_hbm, zsum_hbm, zhist_hbm, psum_hbm, hist_hbm,
                    di_v, mr_v, hist_v, ssum):
        cid = lax.axis_index("c")
        sid = lax.axis_index("s")
        wid = sid * NC + cid
        row0 = sid * NPT
        pltpu.sync_copy(zsum_hbm, ssum.at[pl.ds(row0, NPT)])
        pltpu.sync_copy(zhist_hbm, hist_v)
        plsc.subcore_barrier()

        base = wid * EPW

        def body(k, carry):
            off = pl.multiple_of(base + k * CH, 8)
            pltpu.sync_copy(dst_hbm.at[pl.ds(off, CH)], di_v)
            pltpu.sync_copy(msg_hbm.at[pl.ds(off, CH)], mr_v)
            pltpu.sync_copy(mr_v, ssum.at[di_v], add=True)
            for j in range(CH // 16):
                idx = di_v[pl.ds(j * 16, 16)]
                plsc.addupdate_scatter(hist_v, [idx], jnp.ones((16,), jnp.float32))
            return carry

        lax.fori_loop(0, KCH, body, 0)
        plsc.subcore_barrier()
        out0 = pl.multiple_of(cid * NP + row0, 8)
        pltpu.sync_copy(ssum.at[pl.ds(row0, NPT)], psum_hbm.at[pl.ds(out0, NPT)])
        h0 = pl.multiple_of(wid * NP, 8)
        pltpu.sync_copy(hist_v, hist_hbm.at[pl.ds(h0, NP)])

    return _sc_scatter


BE = 256


def _msg_body(et_ref, hs_ref, hd_ref, wct_ref, bc_ref, m_ref, out_ref, g_ref):
    hd = hd_ref[...]
    hs = hs_ref[...].astype(jnp.bfloat16)
    coef = jnp.dot(hd, wct_ref[...], preferred_element_type=jnp.float32)
    coef = coef + bc_ref[...]
    coef = jnp.where(coef >= 0.0, coef, 0.2 * coef)
    rel = lax.broadcasted_iota(jnp.int32, (BE, RM), 1) // MEM
    coef = jnp.where(rel == et_ref[...], coef, 0.0).astype(jnp.bfloat16)
    for j in range(RM):
        g_ref[:, j * IN_FEATS:(j + 1) * IN_FEATS] = coef[:, j:j + 1] * hs
    out_ref[...] = jnp.dot(g_ref[...], m_ref[...],
                           preferred_element_type=jnp.float32)


def _tc_msg(et2, hs, hd, wc_t, bc_all, m_flat):
    grid = (EP // BE,)
    return pl.pallas_call(
        _msg_body,
        grid=grid,
        in_specs=[
            pl.BlockSpec((BE, 1), lambda i: (i, 0)),
            pl.BlockSpec((BE, IN_FEATS), lambda i: (i, 0)),
            pl.BlockSpec((BE, IN_FEATS), lambda i: (i, 0)),
            pl.BlockSpec((IN_FEATS, RM), lambda i: (0, 0)),
            pl.BlockSpec((1, RM), lambda i: (0, 0)),
            pl.BlockSpec((RM * IN_FEATS, OUT_FEATS), lambda i: (0, 0)),
        ],
        out_specs=pl.BlockSpec((BE, OUT_FEATS), lambda i: (i, 0)),
        out_shape=jax.ShapeDtypeStruct((EP, OUT_FEATS), jnp.float32),
        scratch_shapes=[pltpu.VMEM((BE, RM * IN_FEATS), jnp.bfloat16)],
        compiler_params=pltpu.CompilerParams(
            dimension_semantics=("arbitrary",)),
    )(et2, hs, hd, wc_t, bc_all, m_flat)


BN = 1000


def _fin_body(p0_ref, p1_ref, ht_ref, x_ref, wcnt_ref, bcn_ref,
              mn_ref, hb_ref, g_ref, b_ref, out_ref, gn_ref):
    sums = p0_ref[...] + p1_ref[...]
    cnt = jnp.sum(ht_ref[...], axis=1, keepdims=True)
    rep = jnp.where(cnt > 0.0, sums / jnp.maximum(cnt, 1.0), 0.0)
    mu = jnp.mean(rep, axis=-1, keepdims=True)
    ctr = rep - mu
    var = jnp.mean(ctr * ctr, axis=-1, keepdims=True)
    rep = ctr * lax.rsqrt(var + 1e-5) * g_ref[...] + b_ref[...] + hb_ref[...]
    xb = x_ref[...]
    coefn = jnp.dot(xb, wcnt_ref[...], preferred_element_type=jnp.float32)
    coefn = coefn + bcn_ref[...]
    coefn = jnp.where(coefn >= 0.0, coefn, 0.2 * coefn)
    for m in range(MEM):
        gn_ref[:, m * IN_FEATS:(m + 1) * IN_FEATS] = coefn[:, m:m + 1] * xb
    rep = rep + jnp.dot(gn_ref[...], mn_ref[...],
                        preferred_element_type=jnp.float32)
    out_ref[...] = jnp.where(rep >= 0.0, rep, 0.2 * rep)


def _tc_finalize(p0, p1, hist_t, x, wcn_t, bcn, mn_flat, h_bias, ln_g, ln_b):
    grid = (N // BN,)
    return pl.pallas_call(
        _fin_body,
        grid=grid,
        in_specs=[
            pl.BlockSpec((BN, OUT_FEATS), lambda i: (i, 0)),
            pl.BlockSpec((BN, OUT_FEATS), lambda i: (i, 0)),
            pl.BlockSpec((BN, NW), lambda i: (i, 0)),
            pl.BlockSpec((BN, IN_FEATS), lambda i: (i, 0)),
            pl.BlockSpec((IN_FEATS, MEM), lambda i: (0, 0)),
            pl.BlockSpec((1, MEM), lambda i: (0, 0)),
            pl.BlockSpec((MEM * IN_FEATS, OUT_FEATS), lambda i: (0, 0)),
            pl.BlockSpec((1, OUT_FEATS), lambda i: (0, 0)),
            pl.BlockSpec((1, OUT_FEATS), lambda i: (0, 0)),
            pl.BlockSpec((1, OUT_FEATS), lambda i: (0, 0)),
        ],
        out_specs=pl.BlockSpec((BN, OUT_FEATS), lambda i: (i, 0)),
        out_shape=jax.ShapeDtypeStruct((N, OUT_FEATS), jnp.float32),
        scratch_shapes=[pltpu.VMEM((BN, MEM * IN_FEATS), jnp.float32)],
        compiler_params=pltpu.CompilerParams(
            dimension_semantics=("arbitrary",)),
    )(p0, p1, hist_t, x, wcn_t, bcn, mn_flat, h_bias, ln_g, ln_b)


def kernel(x, edge_index, edge_type, wc_rel, bc_rel, ww_rel,
           wc_node, bc_node, ww_node, h_bias, ln_gamma, ln_beta):
    src = edge_index[0]
    dst = edge_index[1]

    m_flat = ww_rel.reshape(NUM_RELS, OUT_FEATS, IN_FEATS, MEM)
    m_flat = m_flat.transpose(0, 3, 2, 1).reshape(RM * IN_FEATS, OUT_FEATS)
    wc_t = wc_rel.reshape(RM, IN_FEATS).T
    bc_all = bc_rel.reshape(1, RM)
    mn_flat = ww_node.reshape(OUT_FEATS, IN_FEATS, MEM)
    mn_flat = mn_flat.transpose(2, 1, 0).reshape(MEM * IN_FEATS, OUT_FEATS)
    wcn_t = wc_node.T
    bcn = bc_node.reshape(1, MEM)

    pad_i = jnp.zeros((EP - E,), jnp.int32)
    src_p = jnp.concatenate([src, pad_i])
    dst_gp = jnp.concatenate([dst, pad_i])
    dst_sp = jnp.concatenate([dst, pad_i + (NP - 1)])
    et_p = jnp.concatenate([edge_type, pad_i])

    hs, hd = _build_sc_gather()(x, src_p, dst_gp)

    msg = _tc_msg(et_p.reshape(EP, 1), hs, hd,
                  wc_t, bc_all, m_flat.astype(jnp.bfloat16))

    zsum = jnp.zeros((NPT, OUT_FEATS), jnp.float32)
    zhist = jnp.zeros((NP,), jnp.float32)
    psum, hist = _build_sc_scatter()(msg, dst_sp, zsum, zhist)
    psum = psum.reshape(NC, NP, OUT_FEATS)
    hist_t = hist.reshape(NW, NP)[:, :N].T

    return _tc_finalize(psum[0, :N], psum[1, :N], hist_t, x,
                        wcn_t, bcn, mn_flat,
                        h_bias.reshape(1, OUT_FEATS),
                        ln_gamma.reshape(1, OUT_FEATS),
                        ln_beta.reshape(1, OUT_FEATS))


# speedup vs baseline: 1.0253x; 1.0253x over previous
import functools

import jax
import jax.numpy as jnp
from jax import lax
from jax.experimental import pallas as pl
from jax.experimental.pallas import tpu as pltpu
from jax.experimental.pallas import tpu_sc as plsc

N = 10000
E = 160000
IN_FEATS = 128
OUT_FEATS = 128
MEM = 8
NUM_RELS = 5
RM = NUM_RELS * MEM

NC = 2
NS = 16
NW = NC * NS
CH = 128
EP = 163840
EPW = EP // NW
KCH = EPW // CH
W32 = 64
K0CH = 58
K1CH = 22
EPWMAX = K0CH * CH
NP = 10240
NPT = NP // NS

@functools.lru_cache(maxsize=None)
def _build_sc_gather():
    mesh = plsc.VectorSubcoreMesh(
        core_axis_name="c", subcore_axis_name="s",
        num_cores=NC, num_subcores=NS)

    @functools.partial(
        pl.kernel,
        out_type=(jax.ShapeDtypeStruct((EP, IN_FEATS), jnp.float32),
                  jax.ShapeDtypeStruct((EP, IN_FEATS), jnp.float32)),
        mesh=mesh,
        scratch_types=[
            pltpu.VMEM((EPWMAX,), jnp.int32),
            pltpu.VMEM((EPWMAX,), jnp.int32),
            pltpu.VMEM((2, CH, IN_FEATS), jnp.float32),
            pltpu.VMEM((2, CH, IN_FEATS), jnp.float32),
            pltpu.SemaphoreType.DMA,
            pltpu.SemaphoreType.DMA,
            pltpu.SemaphoreType.DMA,
            pltpu.SemaphoreType.DMA,
        ],
    )
    def _sc_gather(x_hbm, src_hbm, dst_hbm, hs_hbm, hd_hbm,
                   si_v, di_v, sr_v, dr_v, ss0, ss1, sd0, sd1):
        cid = lax.axis_index("c")
        sid = lax.axis_index("s")
        my_k = jnp.where(cid == 0, K0CH, K1CH)
        base_c = jnp.where(cid == 0, NS * K1CH + sid * K0CH, sid * K1CH)
        base = pl.multiple_of(base_c * CH, 8)
        my_e = my_k * CH
        pltpu.sync_copy(src_hbm.at[pl.ds(base, EPWMAX)], si_v)
        pltpu.sync_copy(dst_hbm.at[pl.ds(base, EPWMAX)], di_v)

        def body(k2, carry):
            o0 = pl.multiple_of(base + (2 * k2) * CH, 8)
            o1 = pl.multiple_of(base + (2 * k2 + 1) * CH, 8)
            c0 = 2 * k2 * CH
            c1 = (2 * k2 + 1) * CH
            cs0 = pltpu.async_copy(x_hbm.at[si_v.at[pl.ds(c0, CH)]],
                                   sr_v.at[0], ss0)
            cd0 = pltpu.async_copy(x_hbm.at[di_v.at[pl.ds(c0, CH)]],
                                   dr_v.at[0], sd0)
            cs1 = pltpu.async_copy(x_hbm.at[si_v.at[pl.ds(c1, CH)]],
                                   sr_v.at[1], ss1)
            cd1 = pltpu.async_copy(x_hbm.at[di_v.at[pl.ds(c1, CH)]],
                                   dr_v.at[1], sd1)
            cs0.wait()
            pltpu.sync_copy(sr_v.at[0], hs_hbm.at[pl.ds(o0, CH)])
            cd0.wait()
            pltpu.sync_copy(dr_v.at[0], hd_hbm.at[pl.ds(o0, CH)])
            cs1.wait()
            pltpu.sync_copy(sr_v.at[1], hs_hbm.at[pl.ds(o1, CH)])
            cd1.wait()
            pltpu.sync_copy(dr_v.at[1], hd_hbm.at[pl.ds(o1, CH)])
            return carry

        lax.fori_loop(0, my_k // 2, body, 0)

    return _sc_gather


@functools.lru_cache(maxsize=None)
def _build_sc_scatter():
    mesh = plsc.VectorSubcoreMesh(
        core_axis_name="c", subcore_axis_name="s",
        num_cores=NC, num_subcores=NS)

    @functools.partial(
        pl.kernel,
        out_type=(jax.ShapeDtypeStruct((NC * NP, OUT_FEATS), jnp.float32),
                  jax.ShapeDtypeStruct((NW * NP,), jnp.float32)),
        mesh=mesh,
        compiler_params=pltpu.CompilerParams(needs_layout_passes=False),
        scratch_types=[
            pltpu.VMEM((CH,), jnp.int32),
            pltpu.VMEM((CH, OUT_FEATS), jnp.float32),
            pltpu.VMEM((NP,), jnp.float32),
            pltpu.VMEM_SHARED((NP, OUT_FEATS), jnp.float32),
        ],
    )
    def _sc_scatter(msg_hbm, dst_hbm, zsum_hbm, zhist_hbm, psum_hbm, hist_hbm,
                    di_v, mr_v, hist_v, ssum):
        cid = lax.axis_index("c")
        sid = lax.axis_index("s")
        wid = sid * NC + cid
        row0 = sid * NPT
        pltpu.sync_copy(zsum_hbm, ssum.at[pl.ds(row0, NPT)])
        pltpu.sync_copy(zhist_hbm, hist_v)
        plsc.subcore_barrier()

        base = wid * EPW

        def body(k, carry):
            off = pl.multiple_of(base + k * CH, 8)
            pltpu.sync_copy(dst_hbm.at[pl.ds(off, CH)], di_v)
            pltpu.sync_copy(msg_hbm.at[pl.ds(off, CH)], mr_v)
            pltpu.sync_copy(mr_v, ssum.at[di_v], add=True)
            for j in range(CH // 16):
                idx = di_v[pl.ds(j * 16, 16)]
                plsc.addupdate_scatter(hist_v, [idx], jnp.ones((16,), jnp.float32))
            return carry

        lax.fori_loop(0, KCH, body, 0)
        plsc.subcore_barrier()
        out0 = pl.multiple_of(cid * NP + row0, 8)
        pltpu.sync_copy(ssum.at[pl.ds(row0, NPT)], psum_hbm.at[pl.ds(out0, NPT)])
        h0 = pl.multiple_of(wid * NP, 8)
        pltpu.sync_copy(hist_v, hist_hbm.at[pl.ds(h0, NP)])

    return _sc_scatter


BE = 256


def _msg_body(et_ref, hs_ref, hd_ref, wct_ref, bc_ref, m_ref, out_ref, g_ref):
    hd = hd_ref[...]
    hs = hs_ref[...].astype(jnp.bfloat16)
    coef = jnp.dot(hd, wct_ref[...], preferred_element_type=jnp.float32)
    coef = coef + bc_ref[...]
    coef = jnp.where(coef >= 0.0, coef, 0.2 * coef)
    rel = lax.broadcasted_iota(jnp.int32, (BE, RM), 1) // MEM
    coef = jnp.where(rel == et_ref[...], coef, 0.0).astype(jnp.bfloat16)
    for j in range(RM):
        g_ref[:, j * IN_FEATS:(j + 1) * IN_FEATS] = coef[:, j:j + 1] * hs
    out_ref[...] = jnp.dot(g_ref[...], m_ref[...],
                           preferred_element_type=jnp.float32)


def _tc_msg(et2, hs, hd, wc_t, bc_all, m_flat):
    grid = (EP // BE,)
    return pl.pallas_call(
        _msg_body,
        grid=grid,
        in_specs=[
            pl.BlockSpec((BE, 1), lambda i: (i, 0)),
            pl.BlockSpec((BE, IN_FEATS), lambda i: (i, 0)),
            pl.BlockSpec((BE, IN_FEATS), lambda i: (i, 0)),
            pl.BlockSpec((IN_FEATS, RM), lambda i: (0, 0)),
            pl.BlockSpec((1, RM), lambda i: (0, 0)),
            pl.BlockSpec((RM * IN_FEATS, OUT_FEATS), lambda i: (0, 0)),
        ],
        out_specs=pl.BlockSpec((BE, OUT_FEATS), lambda i: (i, 0)),
        out_shape=jax.ShapeDtypeStruct((EP, OUT_FEATS), jnp.float32),
        scratch_shapes=[pltpu.VMEM((BE, RM * IN_FEATS), jnp.bfloat16)],
        compiler_params=pltpu.CompilerParams(
            dimension_semantics=("arbitrary",)),
    )(et2, hs, hd, wc_t, bc_all, m_flat)


BN = 1000


def _fin_body(p0_ref, p1_ref, ht_ref, x_ref, wcnt_ref, bcn_ref,
              mn_ref, hb_ref, g_ref, b_ref, out_ref, gn_ref):
    sums = p0_ref[...] + p1_ref[...]
    cnt = jnp.sum(ht_ref[...], axis=1, keepdims=True)
    rep = jnp.where(cnt > 0.0, sums / jnp.maximum(cnt, 1.0), 0.0)
    mu = jnp.mean(rep, axis=-1, keepdims=True)
    ctr = rep - mu
    var = jnp.mean(ctr * ctr, axis=-1, keepdims=True)
    rep = ctr * lax.rsqrt(var + 1e-5) * g_ref[...] + b_ref[...] + hb_ref[...]
    xb = x_ref[...]
    coefn = jnp.dot(xb, wcnt_ref[...], preferred_element_type=jnp.float32)
    coefn = coefn + bcn_ref[...]
    coefn = jnp.where(coefn >= 0.0, coefn, 0.2 * coefn)
    for m in range(MEM):
        gn_ref[:, m * IN_FEATS:(m + 1) * IN_FEATS] = coefn[:, m:m + 1] * xb
    rep = rep + jnp.dot(gn_ref[...], mn_ref[...],
                        preferred_element_type=jnp.float32)
    out_ref[...] = jnp.where(rep >= 0.0, rep, 0.2 * rep)


def _tc_finalize(p0, p1, hist_t, x, wcn_t, bcn, mn_flat, h_bias, ln_g, ln_b):
    grid = (N // BN,)
    return pl.pallas_call(
        _fin_body,
        grid=grid,
        in_specs=[
            pl.BlockSpec((BN, OUT_FEATS), lambda i: (i, 0)),
            pl.BlockSpec((BN, OUT_FEATS), lambda i: (i, 0)),
            pl.BlockSpec((BN, NW), lambda i: (i, 0)),
            pl.BlockSpec((BN, IN_FEATS), lambda i: (i, 0)),
            pl.BlockSpec((IN_FEATS, MEM), lambda i: (0, 0)),
            pl.BlockSpec((1, MEM), lambda i: (0, 0)),
            pl.BlockSpec((MEM * IN_FEATS, OUT_FEATS), lambda i: (0, 0)),
            pl.BlockSpec((1, OUT_FEATS), lambda i: (0, 0)),
            pl.BlockSpec((1, OUT_FEATS), lambda i: (0, 0)),
            pl.BlockSpec((1, OUT_FEATS), lambda i: (0, 0)),
        ],
        out_specs=pl.BlockSpec((BN, OUT_FEATS), lambda i: (i, 0)),
        out_shape=jax.ShapeDtypeStruct((N, OUT_FEATS), jnp.float32),
        scratch_shapes=[pltpu.VMEM((BN, MEM * IN_FEATS), jnp.float32)],
        compiler_params=pltpu.CompilerParams(
            dimension_semantics=("arbitrary",)),
    )(p0, p1, hist_t, x, wcn_t, bcn, mn_flat, h_bias, ln_g, ln_b)


def kernel(x, edge_index, edge_type, wc_rel, bc_rel, ww_rel,
           wc_node, bc_node, ww_node, h_bias, ln_gamma, ln_beta):
    src = edge_index[0]
    dst = edge_index[1]

    m_flat = ww_rel.reshape(NUM_RELS, OUT_FEATS, IN_FEATS, MEM)
    m_flat = m_flat.transpose(0, 3, 2, 1).reshape(RM * IN_FEATS, OUT_FEATS)
    wc_t = wc_rel.reshape(RM, IN_FEATS).T
    bc_all = bc_rel.reshape(1, RM)
    mn_flat = ww_node.reshape(OUT_FEATS, IN_FEATS, MEM)
    mn_flat = mn_flat.transpose(2, 1, 0).reshape(MEM * IN_FEATS, OUT_FEATS)
    wcn_t = wc_node.T
    bcn = bc_node.reshape(1, MEM)

    pad_i = jnp.zeros((EP - E,), jnp.int32)
    src_p = jnp.concatenate([src, pad_i])
    dst_gp = jnp.concatenate([dst, pad_i])
    dst_sp = jnp.concatenate([dst, pad_i + (NP - 1)])
    et_p = jnp.concatenate([edge_type, pad_i])

    hs, hd = _build_sc_gather()(x, src_p, dst_gp)

    msg = _tc_msg(et_p.reshape(EP, 1), hs, hd,
                  wc_t, bc_all, m_flat.astype(jnp.bfloat16))

    zsum = jnp.zeros((NPT, OUT_FEATS), jnp.float32)
    zhist = jnp.zeros((NP,), jnp.float32)
    psum, hist = _build_sc_scatter()(msg, dst_sp, zsum, zhist)
    psum = psum.reshape(NC, NP, OUT_FEATS)
    hist_t = hist.reshape(NW, NP)[:, :N].T

    return _tc_finalize(psum[0, :N], psum[1, :N], hist_t, x,
                        wcn_t, bcn, mn_flat,
                        h_bias.reshape(1, OUT_FEATS),
                        ln_gamma.reshape(1, OUT_FEATS),
                        ln_beta.reshape(1, OUT_FEATS))


# speedup vs baseline: 1.0676x; 1.0413x over previous
import functools

import jax
import jax.numpy as jnp
from jax import lax
from jax.experimental import pallas as pl
from jax.experimental.pallas import tpu as pltpu
from jax.experimental.pallas import tpu_sc as plsc

N = 10000
E = 160000
IN_FEATS = 128
OUT_FEATS = 128
MEM = 8
NUM_RELS = 5
RM = NUM_RELS * MEM

NC = 2
NS = 16
NW = NC * NS
CH = 128
EP = 163840
EPW = EP // NW
KCH = EPW // CH
W32 = 64
K0CH = 58
K1CH = 22
EPWMAX = K0CH * CH
NP = 10240
NPT = NP // NS

@functools.lru_cache(maxsize=None)
def _build_sc_gather():
    mesh = plsc.VectorSubcoreMesh(
        core_axis_name="c", subcore_axis_name="s",
        num_cores=NC, num_subcores=NS)

    @functools.partial(
        pl.kernel,
        out_type=(jax.ShapeDtypeStruct((EP, IN_FEATS), jnp.float32),
                  jax.ShapeDtypeStruct((EP, IN_FEATS), jnp.float32)),
        mesh=mesh,
        scratch_types=[
            pltpu.VMEM((EPWMAX,), jnp.int32),
            pltpu.VMEM((EPWMAX,), jnp.int32),
            pltpu.VMEM((2, CH, IN_FEATS), jnp.float32),
            pltpu.VMEM((2, CH, IN_FEATS), jnp.float32),
            pltpu.SemaphoreType.DMA,
            pltpu.SemaphoreType.DMA,
            pltpu.SemaphoreType.DMA,
            pltpu.SemaphoreType.DMA,
        ],
    )
    def _sc_gather(x_hbm, src_hbm, dst_hbm, hs_hbm, hd_hbm,
                   si_v, di_v, sr_v, dr_v, ss0, ss1, sd0, sd1):
        cid = lax.axis_index("c")
        sid = lax.axis_index("s")
        my_k = jnp.where(cid == 0, K0CH, K1CH)
        base_c = jnp.where(cid == 0, NS * K1CH + sid * K0CH, sid * K1CH)
        base = pl.multiple_of(base_c * CH, 8)
        my_e = my_k * CH
        pltpu.sync_copy(src_hbm.at[pl.ds(base, EPWMAX)], si_v)
        pltpu.sync_copy(dst_hbm.at[pl.ds(base, EPWMAX)], di_v)

        def body(k2, carry):
            o0 = pl.multiple_of(base + (2 * k2) * CH, 8)
            o1 = pl.multiple_of(base + (2 * k2 + 1) * CH, 8)
            c0 = 2 * k2 * CH
            c1 = (2 * k2 + 1) * CH
            cs0 = pltpu.async_copy(x_hbm.at[si_v.at[pl.ds(c0, CH)]],
                                   sr_v.at[0], ss0)
            cd0 = pltpu.async_copy(x_hbm.at[di_v.at[pl.ds(c0, CH)]],
                                   dr_v.at[0], sd0)
            cs1 = pltpu.async_copy(x_hbm.at[si_v.at[pl.ds(c1, CH)]],
                                   sr_v.at[1], ss1)
            cd1 = pltpu.async_copy(x_hbm.at[di_v.at[pl.ds(c1, CH)]],
                                   dr_v.at[1], sd1)
            cs0.wait()
            pltpu.sync_copy(sr_v.at[0], hs_hbm.at[pl.ds(o0, CH)])
            cd0.wait()
            pltpu.sync_copy(dr_v.at[0], hd_hbm.at[pl.ds(o0, CH)])
            cs1.wait()
            pltpu.sync_copy(sr_v.at[1], hs_hbm.at[pl.ds(o1, CH)])
            cd1.wait()
            pltpu.sync_copy(dr_v.at[1], hd_hbm.at[pl.ds(o1, CH)])
            return carry

        lax.fori_loop(0, my_k // 2, body, 0)

    return _sc_gather


@functools.lru_cache(maxsize=None)
def _build_sc_scatter():
    mesh = plsc.VectorSubcoreMesh(
        core_axis_name="c", subcore_axis_name="s",
        num_cores=NC, num_subcores=NS)

    @functools.partial(
        pl.kernel,
        out_type=(jax.ShapeDtypeStruct((NC * NP, OUT_FEATS), jnp.float32),
                  jax.ShapeDtypeStruct((NW * NP,), jnp.float32)),
        mesh=mesh,
        compiler_params=pltpu.CompilerParams(needs_layout_passes=False),
        scratch_types=[
            pltpu.VMEM((2, CH), jnp.int32),
            pltpu.VMEM((2, CH, OUT_FEATS), jnp.float32),
            pltpu.VMEM((NP,), jnp.float32),
            pltpu.VMEM_SHARED((NP, OUT_FEATS), jnp.float32),
            pltpu.SemaphoreType.DMA,
            pltpu.SemaphoreType.DMA,
        ],
    )
    def _sc_scatter(msg_hbm, dst_hbm, zsum_hbm, zhist_hbm, psum_hbm, hist_hbm,
                    di_v, mr_v, hist_v, ssum, sm0, sm1):
        cid = lax.axis_index("c")
        sid = lax.axis_index("s")
        wid = sid * NC + cid
        row0 = sid * NPT
        pltpu.sync_copy(zsum_hbm, ssum.at[pl.ds(row0, NPT)])
        pltpu.sync_copy(zhist_hbm, hist_v)
        plsc.subcore_barrier()

        base = wid * EPW

        def body(k2, carry):
            o0 = pl.multiple_of(base + (2 * k2) * CH, 8)
            o1 = pl.multiple_of(base + (2 * k2 + 1) * CH, 8)
            pltpu.sync_copy(dst_hbm.at[pl.ds(o0, CH)], di_v.at[0])
            pltpu.sync_copy(dst_hbm.at[pl.ds(o1, CH)], di_v.at[1])
            c0 = pltpu.async_copy(msg_hbm.at[pl.ds(o0, CH)], mr_v.at[0], sm0)
            c1 = pltpu.async_copy(msg_hbm.at[pl.ds(o1, CH)], mr_v.at[1], sm1)
            c0.wait()
            pltpu.sync_copy(mr_v.at[0], ssum.at[di_v.at[0]], add=True)
            for j in range(CH // 16):
                idx = di_v[0, pl.ds(j * 16, 16)]
                plsc.addupdate_scatter(hist_v, [idx], jnp.ones((16,), jnp.float32))
            c1.wait()
            pltpu.sync_copy(mr_v.at[1], ssum.at[di_v.at[1]], add=True)
            for j in range(CH // 16):
                idx = di_v[1, pl.ds(j * 16, 16)]
                plsc.addupdate_scatter(hist_v, [idx], jnp.ones((16,), jnp.float32))
            return carry

        lax.fori_loop(0, KCH // 2, body, 0)
        plsc.subcore_barrier()
        out0 = pl.multiple_of(cid * NP + row0, 8)
        pltpu.sync_copy(ssum.at[pl.ds(row0, NPT)], psum_hbm.at[pl.ds(out0, NPT)])
        h0 = pl.multiple_of(wid * NP, 8)
        pltpu.sync_copy(hist_v, hist_hbm.at[pl.ds(h0, NP)])

    return _sc_scatter


BE = 512


def _msg_body(et_ref, hs_ref, hd_ref, wct_ref, bc_ref, m_ref, out_ref, g_ref):
    hd = hd_ref[...]
    hs = hs_ref[...].astype(jnp.bfloat16)
    coef = jnp.dot(hd, wct_ref[...], preferred_element_type=jnp.float32)
    coef = coef + bc_ref[...]
    coef = jnp.where(coef >= 0.0, coef, 0.2 * coef)
    rel = lax.broadcasted_iota(jnp.int32, (BE, RM), 1) // MEM
    coef = jnp.where(rel == et_ref[...], coef, 0.0).astype(jnp.bfloat16)
    for j in range(RM):
        g_ref[:, j * IN_FEATS:(j + 1) * IN_FEATS] = coef[:, j:j + 1] * hs
    out_ref[...] = jnp.dot(g_ref[...], m_ref[...],
                           preferred_element_type=jnp.float32)


def _tc_msg(et2, hs, hd, wc_t, bc_all, m_flat):
    grid = (EP // BE,)
    return pl.pallas_call(
        _msg_body,
        grid=grid,
        in_specs=[
            pl.BlockSpec((BE, 1), lambda i: (i, 0)),
            pl.BlockSpec((BE, IN_FEATS), lambda i: (i, 0)),
            pl.BlockSpec((BE, IN_FEATS), lambda i: (i, 0)),
            pl.BlockSpec((IN_FEATS, RM), lambda i: (0, 0)),
            pl.BlockSpec((1, RM), lambda i: (0, 0)),
            pl.BlockSpec((RM * IN_FEATS, OUT_FEATS), lambda i: (0, 0)),
        ],
        out_specs=pl.BlockSpec((BE, OUT_FEATS), lambda i: (i, 0)),
        out_shape=jax.ShapeDtypeStruct((EP, OUT_FEATS), jnp.float32),
        scratch_shapes=[pltpu.VMEM((BE, RM * IN_FEATS), jnp.bfloat16)],
        compiler_params=pltpu.CompilerParams(
            dimension_semantics=("arbitrary",)),
    )(et2, hs, hd, wc_t, bc_all, m_flat)


BN = 1000


def _fin_body(p0_ref, p1_ref, ht_ref, x_ref, wcnt_ref, bcn_ref,
              mn_ref, hb_ref, g_ref, b_ref, out_ref, gn_ref):
    sums = p0_ref[...] + p1_ref[...]
    cnt = jnp.sum(ht_ref[...], axis=1, keepdims=True)
    rep = jnp.where(cnt > 0.0, sums / jnp.maximum(cnt, 1.0), 0.0)
    mu = jnp.mean(rep, axis=-1, keepdims=True)
    ctr = rep - mu
    var = jnp.mean(ctr * ctr, axis=-1, keepdims=True)
    rep = ctr * lax.rsqrt(var + 1e-5) * g_ref[...] + b_ref[...] + hb_ref[...]
    xb = x_ref[...]
    coefn = jnp.dot(xb, wcnt_ref[...], preferred_element_type=jnp.float32)
    coefn = coefn + bcn_ref[...]
    coefn = jnp.where(coefn >= 0.0, coefn, 0.2 * coefn)
    for m in range(MEM):
        gn_ref[:, m * IN_FEATS:(m + 1) * IN_FEATS] = coefn[:, m:m + 1] * xb
    rep = rep + jnp.dot(gn_ref[...], mn_ref[...],
                        preferred_element_type=jnp.float32)
    out_ref[...] = jnp.where(rep >= 0.0, rep, 0.2 * rep)


def _tc_finalize(p0, p1, hist_t, x, wcn_t, bcn, mn_flat, h_bias, ln_g, ln_b):
    grid = (N // BN,)
    return pl.pallas_call(
        _fin_body,
        grid=grid,
        in_specs=[
            pl.BlockSpec((BN, OUT_FEATS), lambda i: (i, 0)),
            pl.BlockSpec((BN, OUT_FEATS), lambda i: (i, 0)),
            pl.BlockSpec((BN, NW), lambda i: (i, 0)),
            pl.BlockSpec((BN, IN_FEATS), lambda i: (i, 0)),
            pl.BlockSpec((IN_FEATS, MEM), lambda i: (0, 0)),
            pl.BlockSpec((1, MEM), lambda i: (0, 0)),
            pl.BlockSpec((MEM * IN_FEATS, OUT_FEATS), lambda i: (0, 0)),
            pl.BlockSpec((1, OUT_FEATS), lambda i: (0, 0)),
            pl.BlockSpec((1, OUT_FEATS), lambda i: (0, 0)),
            pl.BlockSpec((1, OUT_FEATS), lambda i: (0, 0)),
        ],
        out_specs=pl.BlockSpec((BN, OUT_FEATS), lambda i: (i, 0)),
        out_shape=jax.ShapeDtypeStruct((N, OUT_FEATS), jnp.float32),
        scratch_shapes=[pltpu.VMEM((BN, MEM * IN_FEATS), jnp.float32)],
        compiler_params=pltpu.CompilerParams(
            dimension_semantics=("arbitrary",)),
    )(p0, p1, hist_t, x, wcn_t, bcn, mn_flat, h_bias, ln_g, ln_b)


def kernel(x, edge_index, edge_type, wc_rel, bc_rel, ww_rel,
           wc_node, bc_node, ww_node, h_bias, ln_gamma, ln_beta):
    src = edge_index[0]
    dst = edge_index[1]

    m_flat = ww_rel.reshape(NUM_RELS, OUT_FEATS, IN_FEATS, MEM)
    m_flat = m_flat.transpose(0, 3, 2, 1).reshape(RM * IN_FEATS, OUT_FEATS)
    wc_t = wc_rel.reshape(RM, IN_FEATS).T
    bc_all = bc_rel.reshape(1, RM)
    mn_flat = ww_node.reshape(OUT_FEATS, IN_FEATS, MEM)
    mn_flat = mn_flat.transpose(2, 1, 0).reshape(MEM * IN_FEATS, OUT_FEATS)
    wcn_t = wc_node.T
    bcn = bc_node.reshape(1, MEM)

    pad_i = jnp.zeros((EP - E,), jnp.int32)
    src_p = jnp.concatenate([src, pad_i])
    dst_gp = jnp.concatenate([dst, pad_i])
    dst_sp = jnp.concatenate([dst, pad_i + (NP - 1)])
    et_p = jnp.concatenate([edge_type, pad_i])

    hs, hd = _build_sc_gather()(x, src_p, dst_gp)

    msg = _tc_msg(et_p.reshape(EP, 1), hs, hd,
                  wc_t, bc_all, m_flat.astype(jnp.bfloat16))

    zsum = jnp.zeros((NPT, OUT_FEATS), jnp.float32)
    zhist = jnp.zeros((NP,), jnp.float32)
    psum, hist = _build_sc_scatter()(msg, dst_sp, zsum, zhist)
    psum = psum.reshape(NC, NP, OUT_FEATS)
    hist_t = hist.reshape(NW, NP)[:, :N].T

    return _tc_finalize(psum[0, :N], psum[1, :N], hist_t, x,
                        wcn_t, bcn, mn_flat,
                        h_bias.reshape(1, OUT_FEATS),
                        ln_gamma.reshape(1, OUT_FEATS),
                        ln_beta.reshape(1, OUT_FEATS))


# speedup vs baseline: 1.1218x; 1.0507x over previous
import functools

import jax
import jax.numpy as jnp
from jax import lax
from jax.experimental import pallas as pl
from jax.experimental.pallas import tpu as pltpu
from jax.experimental.pallas import tpu_sc as plsc

N = 10000
E = 160000
IN_FEATS = 128
OUT_FEATS = 128
MEM = 8
NUM_RELS = 5
RM = NUM_RELS * MEM

NC = 2
NS = 16
NW = NC * NS
CH = 128
EP = 163840
EPW = EP // NW
KCH = EPW // CH
K0CH = 58
K1CH = 22
EPWMAX = K0CH * CH
NP = 10240
NPT = NP // NS

@functools.lru_cache(maxsize=None)
def _build_sc_gather():
    mesh = plsc.VectorSubcoreMesh(
        core_axis_name="c", subcore_axis_name="s",
        num_cores=NC, num_subcores=NS)

    @functools.partial(
        pl.kernel,
        out_type=(jax.ShapeDtypeStruct((EP, IN_FEATS), jnp.float32),
                  jax.ShapeDtypeStruct((EP, IN_FEATS), jnp.float32)),
        mesh=mesh,
        scratch_types=[
            pltpu.VMEM((EPWMAX,), jnp.int32),
            pltpu.VMEM((EPWMAX,), jnp.int32),
            pltpu.VMEM((2, CH, IN_FEATS), jnp.float32),
            pltpu.VMEM((2, CH, IN_FEATS), jnp.float32),
            pltpu.SemaphoreType.DMA,
            pltpu.SemaphoreType.DMA,
            pltpu.SemaphoreType.DMA,
            pltpu.SemaphoreType.DMA,
        ],
    )
    def _sc_gather(x_hbm, src_hbm, dst_hbm, hs_hbm, hd_hbm,
                   si_v, di_v, sr_v, dr_v, ss0, ss1, sd0, sd1):
        cid = lax.axis_index("c")
        sid = lax.axis_index("s")
        my_k = jnp.where(cid == 0, K0CH, K1CH)
        base_c = jnp.where(cid == 0, NS * K1CH + sid * K0CH, sid * K1CH)
        base = pl.multiple_of(base_c * CH, 8)
        my_e = my_k * CH
        pltpu.sync_copy(src_hbm.at[pl.ds(base, EPWMAX)], si_v)
        pltpu.sync_copy(dst_hbm.at[pl.ds(base, EPWMAX)], di_v)

        def body(k2, carry):
            o0 = pl.multiple_of(base + (2 * k2) * CH, 8)
            o1 = pl.multiple_of(base + (2 * k2 + 1) * CH, 8)
            c0 = 2 * k2 * CH
            c1 = (2 * k2 + 1) * CH
            cs0 = pltpu.async_copy(x_hbm.at[si_v.at[pl.ds(c0, CH)]],
                                   sr_v.at[0], ss0)
            cd0 = pltpu.async_copy(x_hbm.at[di_v.at[pl.ds(c0, CH)]],
                                   dr_v.at[0], sd0)
            cs1 = pltpu.async_copy(x_hbm.at[si_v.at[pl.ds(c1, CH)]],
                                   sr_v.at[1], ss1)
            cd1 = pltpu.async_copy(x_hbm.at[di_v.at[pl.ds(c1, CH)]],
                                   dr_v.at[1], sd1)
            cs0.wait()
            pltpu.sync_copy(sr_v.at[0], hs_hbm.at[pl.ds(o0, CH)])
            cd0.wait()
            pltpu.sync_copy(dr_v.at[0], hd_hbm.at[pl.ds(o0, CH)])
            cs1.wait()
            pltpu.sync_copy(sr_v.at[1], hs_hbm.at[pl.ds(o1, CH)])
            cd1.wait()
            pltpu.sync_copy(dr_v.at[1], hd_hbm.at[pl.ds(o1, CH)])
            return carry

        lax.fori_loop(0, my_k // 2, body, 0)

    return _sc_gather


@functools.lru_cache(maxsize=None)
def _build_sc_scatter():
    mesh = plsc.VectorSubcoreMesh(
        core_axis_name="c", subcore_axis_name="s",
        num_cores=NC, num_subcores=NS)

    @functools.partial(
        pl.kernel,
        out_type=(jax.ShapeDtypeStruct((NC * NP, OUT_FEATS), jnp.float32),
                  jax.ShapeDtypeStruct((NW * NP,), jnp.float32)),
        mesh=mesh,
        compiler_params=pltpu.CompilerParams(needs_layout_passes=False),
        scratch_types=[
            pltpu.VMEM((2, CH), jnp.int32),
            pltpu.VMEM((2, CH, OUT_FEATS), jnp.float32),
            pltpu.VMEM((NP,), jnp.float32),
            pltpu.VMEM_SHARED((NP, OUT_FEATS), jnp.float32),
            pltpu.SemaphoreType.DMA,
            pltpu.SemaphoreType.DMA,
        ],
    )
    def _sc_scatter(msg_hbm, dst_hbm, zsum_hbm, zhist_hbm, psum_hbm, hist_hbm,
                    di_v, mr_v, hist_v, ssum, sm0, sm1):
        cid = lax.axis_index("c")
        sid = lax.axis_index("s")
        wid = sid * NC + cid
        row0 = sid * NPT
        pltpu.sync_copy(zsum_hbm, ssum.at[pl.ds(row0, NPT)])
        pltpu.sync_copy(zhist_hbm, hist_v)
        plsc.subcore_barrier()

        base = wid * EPW

        def body(k2, carry):
            o0 = pl.multiple_of(base + (2 * k2) * CH, 8)
            o1 = pl.multiple_of(base + (2 * k2 + 1) * CH, 8)
            pltpu.sync_copy(dst_hbm.at[pl.ds(o0, CH)], di_v.at[0])
            pltpu.sync_copy(dst_hbm.at[pl.ds(o1, CH)], di_v.at[1])
            c0 = pltpu.async_copy(msg_hbm.at[pl.ds(o0, CH)], mr_v.at[0], sm0)
            c1 = pltpu.async_copy(msg_hbm.at[pl.ds(o1, CH)], mr_v.at[1], sm1)
            c0.wait()
            pltpu.sync_copy(mr_v.at[0], ssum.at[di_v.at[0]], add=True)
            for j in range(CH // 16):
                idx = di_v[0, pl.ds(j * 16, 16)]
                plsc.addupdate_scatter(hist_v, [idx], jnp.ones((16,), jnp.float32))
            c1.wait()
            pltpu.sync_copy(mr_v.at[1], ssum.at[di_v.at[1]], add=True)
            for j in range(CH // 16):
                idx = di_v[1, pl.ds(j * 16, 16)]
                plsc.addupdate_scatter(hist_v, [idx], jnp.ones((16,), jnp.float32))
            return carry

        lax.fori_loop(0, KCH // 2, body, 0)
        plsc.subcore_barrier()
        out0 = pl.multiple_of(cid * NP + row0, 8)
        pltpu.sync_copy(ssum.at[pl.ds(row0, NPT)], psum_hbm.at[pl.ds(out0, NPT)])
        h0 = pl.multiple_of(wid * NP, 8)
        pltpu.sync_copy(hist_v, hist_hbm.at[pl.ds(h0, NP)])

    return _sc_scatter


BE = 1024


def _msg_body(et_ref, hs_ref, hd_ref, wct_ref, bc_ref, m_ref, out_ref, g_ref):
    hd = hd_ref[...]
    hs = hs_ref[...].astype(jnp.bfloat16)
    coef = jnp.dot(hd, wct_ref[...], preferred_element_type=jnp.float32)
    coef = coef + bc_ref[...]
    coef = jnp.where(coef >= 0.0, coef, 0.2 * coef)
    rel = lax.broadcasted_iota(jnp.int32, (BE, RM), 1) // MEM
    coef = jnp.where(rel == et_ref[...], coef, 0.0).astype(jnp.bfloat16)
    for j in range(RM):
        g_ref[:, j * IN_FEATS:(j + 1) * IN_FEATS] = coef[:, j:j + 1] * hs
    out_ref[...] = jnp.dot(g_ref[...], m_ref[...],
                           preferred_element_type=jnp.float32)


def _tc_msg(et2, hs, hd, wc_t, bc_all, m_flat):
    grid = (EP // BE,)
    return pl.pallas_call(
        _msg_body,
        grid=grid,
        in_specs=[
            pl.BlockSpec((BE, 1), lambda i: (i, 0)),
            pl.BlockSpec((BE, IN_FEATS), lambda i: (i, 0)),
            pl.BlockSpec((BE, IN_FEATS), lambda i: (i, 0)),
            pl.BlockSpec((IN_FEATS, RM), lambda i: (0, 0)),
            pl.BlockSpec((1, RM), lambda i: (0, 0)),
            pl.BlockSpec((RM * IN_FEATS, OUT_FEATS), lambda i: (0, 0)),
        ],
        out_specs=pl.BlockSpec((BE, OUT_FEATS), lambda i: (i, 0)),
        out_shape=jax.ShapeDtypeStruct((EP, OUT_FEATS), jnp.float32),
        scratch_shapes=[pltpu.VMEM((BE, RM * IN_FEATS), jnp.bfloat16)],
        compiler_params=pltpu.CompilerParams(
            dimension_semantics=("arbitrary",)),
    )(et2, hs, hd, wc_t, bc_all, m_flat)


BN = 1000


def _fin_body(p0_ref, p1_ref, ht_ref, x_ref, wcnt_ref, bcn_ref,
              mn_ref, hb_ref, g_ref, b_ref, out_ref, gn_ref):
    sums = p0_ref[...] + p1_ref[...]
    cnt = jnp.sum(ht_ref[...], axis=1, keepdims=True)
    rep = jnp.where(cnt > 0.0, sums / jnp.maximum(cnt, 1.0), 0.0)
    mu = jnp.mean(rep, axis=-1, keepdims=True)
    ctr = rep - mu
    var = jnp.mean(ctr * ctr, axis=-1, keepdims=True)
    rep = ctr * lax.rsqrt(var + 1e-5) * g_ref[...] + b_ref[...] + hb_ref[...]
    xb = x_ref[...]
    coefn = jnp.dot(xb, wcnt_ref[...], preferred_element_type=jnp.float32)
    coefn = coefn + bcn_ref[...]
    coefn = jnp.where(coefn >= 0.0, coefn, 0.2 * coefn)
    for m in range(MEM):
        gn_ref[:, m * IN_FEATS:(m + 1) * IN_FEATS] = coefn[:, m:m + 1] * xb
    rep = rep + jnp.dot(gn_ref[...], mn_ref[...],
                        preferred_element_type=jnp.float32)
    out_ref[...] = jnp.where(rep >= 0.0, rep, 0.2 * rep)


def _tc_finalize(p0, p1, hist_t, x, wcn_t, bcn, mn_flat, h_bias, ln_g, ln_b):
    grid = (N // BN,)
    return pl.pallas_call(
        _fin_body,
        grid=grid,
        in_specs=[
            pl.BlockSpec((BN, OUT_FEATS), lambda i: (i, 0)),
            pl.BlockSpec((BN, OUT_FEATS), lambda i: (i, 0)),
            pl.BlockSpec((BN, NW), lambda i: (i, 0)),
            pl.BlockSpec((BN, IN_FEATS), lambda i: (i, 0)),
            pl.BlockSpec((IN_FEATS, MEM), lambda i: (0, 0)),
            pl.BlockSpec((1, MEM), lambda i: (0, 0)),
            pl.BlockSpec((MEM * IN_FEATS, OUT_FEATS), lambda i: (0, 0)),
            pl.BlockSpec((1, OUT_FEATS), lambda i: (0, 0)),
            pl.BlockSpec((1, OUT_FEATS), lambda i: (0, 0)),
            pl.BlockSpec((1, OUT_FEATS), lambda i: (0, 0)),
        ],
        out_specs=pl.BlockSpec((BN, OUT_FEATS), lambda i: (i, 0)),
        out_shape=jax.ShapeDtypeStruct((N, OUT_FEATS), jnp.float32),
        scratch_shapes=[pltpu.VMEM((BN, MEM * IN_FEATS), jnp.float32)],
        compiler_params=pltpu.CompilerParams(
            dimension_semantics=("arbitrary",)),
    )(p0, p1, hist_t, x, wcn_t, bcn, mn_flat, h_bias, ln_g, ln_b)


def kernel(x, edge_index, edge_type, wc_rel, bc_rel, ww_rel,
           wc_node, bc_node, ww_node, h_bias, ln_gamma, ln_beta):
    src = edge_index[0]
    dst = edge_index[1]

    m_flat = ww_rel.reshape(NUM_RELS, OUT_FEATS, IN_FEATS, MEM)
    m_flat = m_flat.transpose(0, 3, 2, 1).reshape(RM * IN_FEATS, OUT_FEATS)
    wc_t = wc_rel.reshape(RM, IN_FEATS).T
    bc_all = bc_rel.reshape(1, RM)
    mn_flat = ww_node.reshape(OUT_FEATS, IN_FEATS, MEM)
    mn_flat = mn_flat.transpose(2, 1, 0).reshape(MEM * IN_FEATS, OUT_FEATS)
    wcn_t = wc_node.T
    bcn = bc_node.reshape(1, MEM)

    pad_i = jnp.zeros((EP - E,), jnp.int32)
    src_p = jnp.concatenate([src, pad_i])
    dst_gp = jnp.concatenate([dst, pad_i])
    dst_sp = jnp.concatenate([dst, pad_i + (NP - 1)])
    et_p = jnp.concatenate([edge_type, pad_i])

    hs, hd = _build_sc_gather()(x, src_p, dst_gp)

    msg = _tc_msg(et_p.reshape(EP, 1), hs, hd,
                  wc_t, bc_all, m_flat.astype(jnp.bfloat16))

    zsum = jnp.zeros((NPT, OUT_FEATS), jnp.float32)
    zhist = jnp.zeros((NP,), jnp.float32)
    psum, hist = _build_sc_scatter()(msg, dst_sp, zsum, zhist)
    psum = psum.reshape(NC, NP, OUT_FEATS)
    hist_t = hist.reshape(NW, NP)[:, :N].T

    return _tc_finalize(psum[0, :N], psum[1, :N], hist_t, x,
                        wcn_t, bcn, mn_flat,
                        h_bias.reshape(1, OUT_FEATS),
                        ln_gamma.reshape(1, OUT_FEATS),
                        ln_beta.reshape(1, OUT_FEATS))
